# baseline (speedup 1.0000x reference)
import jax
import jax.numpy as jnp
from jax.experimental import pallas as pl
from jax.experimental.pallas import tpu as pltpu

T, D, F, E = 2048, 1024, 2048, 8
NT = 2
TT = T // NT
FT = 1024
NF = F // FT
TAU = 1.0


def _moe_body(x_ref, wr_ref, wg_ref, wu_ref, wd_ref, out_ref, gate_ref):
    e = pl.program_id(1)
    f = pl.program_id(2)
    first = (e == 0) & (f == 0)

    @pl.when(first)
    def _():
        logits = jnp.dot(x_ref[:], wr_ref[:], preferred_element_type=jnp.float32)
        logits = logits / TAU
        m = jnp.max(logits, axis=1, keepdims=True)
        p = jnp.exp(logits - m)
        gate_ref[:] = p / jnp.sum(p, axis=1, keepdims=True)

    xb = x_ref[:]
    onehot = (jax.lax.broadcasted_iota(jnp.int32, (E, 1), 0) == e).astype(
        jnp.float32
    )
    gcol = jnp.dot(gate_ref[:], onehot, preferred_element_type=jnp.float32)
    NC = 8
    C = FT // NC
    part = None
    for c in range(NC):
        wgc = wg_ref[0, :, c * C:(c + 1) * C]
        wuc = wu_ref[0, :, c * C:(c + 1) * C]
        wdc = wd_ref[0, c * C:(c + 1) * C, :]
        g = jnp.dot(xb, wgc, preferred_element_type=jnp.float32)
        u = jnp.dot(xb, wuc, preferred_element_type=jnp.float32)
        act = (g * jax.nn.sigmoid(g)) * u * gcol
        p = jnp.dot(act, wdc, preferred_element_type=jnp.float32)
        part = p if part is None else part + p

    @pl.when(first)
    def _():
        out_ref[:] = part

    @pl.when(~first)
    def _():
        out_ref[:] = out_ref[:] + part


def kernel(x, Wg, Wu, Wd, Wr):
    return pl.pallas_call(
        _moe_body,
        grid=(NT, E, NF),
        in_specs=[
            pl.BlockSpec((TT, D), lambda t, e, f: (t, 0)),
            pl.BlockSpec((D, E), lambda t, e, f: (0, 0)),
            pl.BlockSpec((1, D, FT), lambda t, e, f: (e, 0, f)),
            pl.BlockSpec((1, D, FT), lambda t, e, f: (e, 0, f)),
            pl.BlockSpec((1, FT, D), lambda t, e, f: (e, f, 0)),
        ],
        out_specs=pl.BlockSpec((TT, D), lambda t, e, f: (t, 0)),
        out_shape=jax.ShapeDtypeStruct((T, D), jnp.float32),
        scratch_shapes=[pltpu.VMEM((TT, E), jnp.float32)],
        compiler_params=pltpu.CompilerParams(
            dimension_semantics=("parallel", "arbitrary", "arbitrary")
        ),
    )(x, Wr, Wg, Wu, Wd)


# speedup vs baseline: 1.8285x; 1.8285x over previous
import jax
import jax.numpy as jnp
from jax.experimental import pallas as pl
from jax.experimental.pallas import tpu as pltpu

T, D, F, E = 2048, 1024, 2048, 8
NT = 2
TT = T // NT
FT = 1024
NF = F // FT
TAU = 1.0


def _moe_body(x_ref, wr_ref, wg_ref, wu_ref, wd_ref, out_ref, gate_ref):
    e = pl.program_id(1)
    f = pl.program_id(2)
    first = (e == 0) & (f == 0)

    @pl.when(first)
    def _():
        logits = jnp.dot(x_ref[:], wr_ref[:], preferred_element_type=jnp.float32)
        logits = logits / TAU
        m = jnp.max(logits, axis=1, keepdims=True)
        p = jnp.exp(logits - m)
        gate_ref[:] = p / jnp.sum(p, axis=1, keepdims=True)

    xb = x_ref[:]
    onehot = (jax.lax.broadcasted_iota(jnp.int32, (E, 1), 0) == e).astype(
        jnp.float32
    )
    gcol = jnp.dot(gate_ref[:], onehot, preferred_element_type=jnp.float32)
    NC = 4
    C = FT // NC
    part = None
    for c in range(NC):
        wgc = wg_ref[0, :, c * C:(c + 1) * C]
        wuc = wu_ref[0, :, c * C:(c + 1) * C]
        wdc = wd_ref[0, c * C:(c + 1) * C, :]
        g = jnp.dot(xb, wgc, preferred_element_type=jnp.float32)
        u = jnp.dot(xb, wuc, preferred_element_type=jnp.float32)
        act = (g * jax.nn.sigmoid(g)) * u * gcol
        p = jnp.dot(act, wdc, preferred_element_type=jnp.float32)
        part = p if part is None else part + p

    @pl.when(first)
    def _():
        out_ref[:] = part

    @pl.when(~first)
    def _():
        out_ref[:] = out_ref[:] + part


def kernel(x, Wg, Wu, Wd, Wr):
    return pl.pallas_call(
        _moe_body,
        grid=(NT, E, NF),
        in_specs=[
            pl.BlockSpec((TT, D), lambda t, e, f: (t, 0)),
            pl.BlockSpec((D, E), lambda t, e, f: (0, 0)),
            pl.BlockSpec((1, D, FT), lambda t, e, f: (e, 0, f)),
            pl.BlockSpec((1, D, FT), lambda t, e, f: (e, 0, f)),
            pl.BlockSpec((1, FT, D), lambda t, e, f: (e, f, 0)),
        ],
        out_specs=pl.BlockSpec((TT, D), lambda t, e, f: (t, 0)),
        out_shape=jax.ShapeDtypeStruct((T, D), jnp.float32),
        scratch_shapes=[pltpu.VMEM((TT, E), jnp.float32)],
        compiler_params=pltpu.CompilerParams(
            dimension_semantics=("parallel", "arbitrary", "arbitrary")
        ),
    )(x, Wr, Wg, Wu, Wd)
